# baseline (speedup 1.0000x reference)
import functools

import jax
import jax.numpy as jnp
from jax import lax
from jax.experimental import pallas as pl
from jax.experimental.pallas import tpu as pltpu
from jax.experimental.pallas import tpu_sc as plsc
from jax._src.pallas import mpmd as _pl_mpmd

B = 128
N = 100000
BLK = 2048
NBLK = (N + BLK - 1) // BLK


def _tc_body(x_ref, zeros_ref, idx_ref, maxv_ref, maxi_ref):
  step = pl.program_id(0)

  @pl.when(step == 0)
  def _init():
    maxv_ref[...] = jnp.full((B, 1), -jnp.inf, jnp.float32)
    maxi_ref[...] = jnp.zeros((B, 1), jnp.int32)

  x = x_ref[...]
  col = lax.broadcasted_iota(jnp.int32, (B, BLK), 1) + step * BLK
  x = jnp.where(col < N, x, -jnp.inf)
  bmax = jnp.max(x, axis=1, keepdims=True)
  barg = jnp.min(jnp.where(x == bmax, col, jnp.int32(2**31 - 1)),
                 axis=1, keepdims=True)
  better = bmax > maxv_ref[...]
  maxi_ref[...] = jnp.where(better, barg, maxi_ref[...])
  maxv_ref[...] = jnp.where(better, bmax, maxv_ref[...])
  zeros_ref[...] = jnp.zeros((B, BLK), jnp.float32)

  @pl.when(step == NBLK - 1)
  def _fin():
    idx_ref[...] = maxi_ref[...]


_tc_call = pl.pallas_call(
    _tc_body,
    grid=(NBLK,),
    in_specs=[pl.BlockSpec((B, BLK), lambda i: (0, i))],
    out_specs=[
        pl.BlockSpec((B, BLK), lambda i: (0, i)),
        pl.BlockSpec((B, 1), lambda i: (0, 0)),
    ],
    out_shape=[
        jax.ShapeDtypeStruct((B, N), jnp.float32),
        jax.ShapeDtypeStruct((B, 1), jnp.int32),
    ],
    scratch_shapes=[
        pltpu.VMEM((B, 1), jnp.float32),
        pltpu.VMEM((B, 1), jnp.int32),
    ],
)


def _sc_body(zeros_hbm, idx_hbm, out_hbm, idx_v, flat_v, ones_v, sem):
  del zeros_hbm
  cid = lax.axis_index("c")
  sid = lax.axis_index("s")

  @pl.when((cid == 0) & (sid == 0))
  def _():
    pltpu.sync_copy(idx_hbm, idx_v)
    for i in range(B // 16):
      rows = lax.broadcasted_iota(jnp.int32, (16,), 0) + i * 16
      flat_v[pl.ds(i * 16, 16)] = idx_v[pl.ds(i * 16, 16)] + rows * N
      ones_v[pl.ds(i * 16, 16)] = jnp.full((16,), 1.0, jnp.float32)
    pltpu.async_copy(ones_v, out_hbm.at[flat_v], sem).wait()


@functools.cache
def _get_sc_call():
  return _pl_mpmd._mpmd_map(
      [(plsc.VectorSubcoreMesh(core_axis_name="c", subcore_axis_name="s"),
        _sc_body)],
      jax.ShapeDtypeStruct((B * N,), jnp.float32),
      input_output_aliases={0: 0},
      scratch_types=[
          pltpu.VMEM((B,), jnp.int32),
          pltpu.VMEM((B,), jnp.int32),
          pltpu.VMEM((B,), jnp.float32),
          pltpu.SemaphoreType.DMA,
      ],
  )


def kernel(sample):
  zeros2d, idx = _tc_call(sample)
  out_flat = _get_sc_call()(zeros2d.reshape(B * N), idx.reshape(B))
  return out_flat.reshape(B, N)


# speedup vs baseline: 1.7390x; 1.7390x over previous
import functools

import jax
import jax.numpy as jnp
from jax import lax
from jax.experimental import pallas as pl
from jax.experimental.pallas import tpu as pltpu
from jax.experimental.pallas import tpu_sc as plsc
from jax._src.pallas import mpmd as _pl_mpmd

B = 128
N = 100000
BLK = 2048
NBLK = (N + BLK - 1) // BLK
SCCOLS = 48 * BLK
NGRP = B // 8


def _tc_body(x_ref, idx_ref, maxv_ref, maxi_ref):
  step = pl.program_id(0)

  @pl.when(step == 0)
  def _init():
    maxv_ref[...] = jnp.full((B, 1), -jnp.inf, jnp.float32)
    maxi_ref[...] = jnp.zeros((B, 1), jnp.int32)

  x = x_ref[...]
  col = lax.broadcasted_iota(jnp.int32, (B, BLK), 1) + step * BLK
  x = jnp.where(col < N, x, -jnp.inf)
  bmax = jnp.max(x, axis=1, keepdims=True)
  barg = jnp.min(jnp.where(x == bmax, col, jnp.int32(2**31 - 1)),
                 axis=1, keepdims=True)
  better = bmax > maxv_ref[...]
  maxi_ref[...] = jnp.where(better, barg, maxi_ref[...])
  maxv_ref[...] = jnp.where(better, bmax, maxv_ref[...])

  @pl.when(step == NBLK - 1)
  def _fin():
    idx_ref[...] = maxi_ref[...]


_tc_call = pl.pallas_call(
    _tc_body,
    grid=(NBLK,),
    in_specs=[pl.BlockSpec((B, BLK), lambda i: (0, i))],
    out_specs=[pl.BlockSpec((B, 1), lambda i: (0, 0))],
    out_shape=[jax.ShapeDtypeStruct((B, 1), jnp.int32)],
    scratch_shapes=[
        pltpu.VMEM((B, 1), jnp.float32),
        pltpu.VMEM((B, 1), jnp.int32),
    ],
)


def _sc_zero_body(out_hbm, zbuf, sem):
  wid = lax.axis_index("s") * 2 + lax.axis_index("c")
  grp = wid // 2
  part = wid % 2

  for i in range(8):
    def _zero(k, carry, i=i):
      zbuf[i, pl.ds(k * 16, 16)] = jnp.zeros((16,), jnp.float32)
      return carry
    lax.fori_loop(0, BLK // 16, _zero, 0)

  r0 = pl.multiple_of(grp * 8, 8)
  copies = []
  for t in range(24):
    off = pl.multiple_of(part * (24 * BLK) + t * BLK, BLK)
    copies.append(pltpu.async_copy(
        zbuf, out_hbm.at[pl.ds(r0, 8), pl.ds(off, BLK)], sem))
  for c in copies:
    c.wait()


def _sc_fix_body(zeros_hbm, idx_hbm, out_hbm, idx_v, tbuf, sem):
  del zeros_hbm
  del sem
  wid = lax.axis_index("s") * 2 + lax.axis_index("c")

  @pl.when(wid < NGRP)
  def _():
    grp = wid
    r0 = pl.multiple_of(grp * 8, 8)
    pltpu.sync_copy(idx_hbm, idx_v)
    lanes = lax.broadcasted_iota(jnp.int32, (16,), 0)
    chunk = idx_v[pl.ds(pl.multiple_of((grp // 2) * 16, 16), 16)]
    chunk_f = chunk.astype(jnp.float32)
    for j in range(8):
      lane = (grp % 2) * 8 + j
      s_f = jnp.max(jnp.where(lanes == lane, chunk_f, jnp.float32(-1.0)))
      s = s_f.astype(jnp.int32)

      @pl.when(s < SCCOLS)
      def _row():
        c0 = pl.multiple_of(lax.bitwise_and(s, jnp.int32(-128)), 128)
        sub = pl.multiple_of(lax.bitwise_and(s - c0, jnp.int32(-16)), 16)
        dst = out_hbm.at[pl.ds(r0, 8), pl.ds(c0, 128)]
        pltpu.sync_copy(dst, tbuf)
        v = tbuf[j, pl.ds(sub, 16)]
        tbuf[j, pl.ds(sub, 16)] = jnp.where(lanes + c0 + sub == s,
                                            jnp.float32(1.0), v)
        pltpu.sync_copy(tbuf, dst)


def _tc_tail_body(cur_ref, idx_ref, out_ref):
  del cur_ref
  col = lax.broadcasted_iota(jnp.int32, (B, BLK), 1) + SCCOLS
  out_ref[...] = jnp.where(col == idx_ref[...], jnp.float32(1.0),
                           jnp.float32(0.0))


_tc_tail_call = pl.pallas_call(
    _tc_tail_body,
    grid=(1,),
    in_specs=[
        pl.BlockSpec(memory_space=pltpu.MemorySpace.HBM),
        pl.BlockSpec((B, 1), lambda i: (0, 0)),
    ],
    out_specs=[pl.BlockSpec((B, BLK), lambda i: (0, 48))],
    out_shape=[jax.ShapeDtypeStruct((B, N), jnp.float32)],
    input_output_aliases={0: 0},
)


@functools.cache
def _get_sc_calls():
  mesh = plsc.VectorSubcoreMesh(core_axis_name="c", subcore_axis_name="s")
  params = pltpu.CompilerParams(needs_layout_passes=False)
  zero_call = _pl_mpmd._mpmd_map(
      [(mesh, _sc_zero_body)],
      jax.ShapeDtypeStruct((B, N), jnp.float32),
      compiler_params=params,
      scratch_types=[
          pltpu.VMEM((8, BLK), jnp.float32),
          pltpu.SemaphoreType.DMA,
      ],
  )
  fix_call = _pl_mpmd._mpmd_map(
      [(mesh, _sc_fix_body)],
      jax.ShapeDtypeStruct((B, N), jnp.float32),
      input_output_aliases={0: 0},
      compiler_params=params,
      scratch_types=[
          pltpu.VMEM((B,), jnp.int32),
          pltpu.VMEM((8, 128), jnp.float32),
          pltpu.SemaphoreType.DMA,
      ],
  )
  return zero_call, fix_call


def kernel(sample):
  zero_call, fix_call = _get_sc_calls()
  idx2d = _tc_call(sample)[0]
  zeros = zero_call()
  fixed = fix_call(zeros, idx2d.reshape(B))
  return _tc_tail_call(fixed, idx2d)[0]


# speedup vs baseline: 4.4712x; 2.5711x over previous
import functools

import jax
import jax.numpy as jnp
from jax import lax
from jax.experimental import pallas as pl
from jax.experimental.pallas import tpu as pltpu
from jax.experimental.pallas import tpu_sc as plsc
from jax._src.pallas import mpmd as _pl_mpmd

B = 128
N = 100000
BLKR = 8192
NBLK = (N + BLKR - 1) // BLKR

NTILES = 32
ZROWS = 256
NCHUNK = (N + ZROWS - 1) // ZROWS
ZLAST = (N - ZROWS) // 8 * 8
ZPT = (NCHUNK + NTILES - 1) // NTILES
OWN = 3200


def _tc_body(x_ref, idx_ref, maxv_ref, maxi_ref):
  step = pl.program_id(0)

  @pl.when(step == 0)
  def _init():
    maxv_ref[...] = jnp.full((1, B), -jnp.inf, jnp.float32)
    maxi_ref[...] = jnp.zeros((1, B), jnp.int32)

  x = x_ref[...]
  row = lax.broadcasted_iota(jnp.int32, (BLKR, B), 0) + step * BLKR
  x = jnp.where(row < N, x, -jnp.inf)
  bmax = jnp.max(x, axis=0, keepdims=True)
  barg = jnp.min(jnp.where(x == bmax, row, jnp.int32(2**31 - 1)),
                 axis=0, keepdims=True)
  better = bmax > maxv_ref[...]
  maxi_ref[...] = jnp.where(better, barg, maxi_ref[...])
  maxv_ref[...] = jnp.where(better, bmax, maxv_ref[...])

  @pl.when(step == NBLK - 1)
  def _fin():
    idx_ref[...] = maxi_ref[...]


_tc_call = pl.pallas_call(
    _tc_body,
    grid=(NBLK,),
    in_specs=[pl.BlockSpec((BLKR, B), lambda i: (i, 0))],
    out_specs=[pl.BlockSpec((1, B), lambda i: (0, 0))],
    out_shape=[jax.ShapeDtypeStruct((1, B), jnp.int32)],
    scratch_shapes=[
        pltpu.VMEM((1, B), jnp.float32),
        pltpu.VMEM((1, B), jnp.int32),
    ],
)


def _sc_zero_body(out_hbm, zbuf, sem):
  wid = lax.axis_index("s") * 2 + lax.axis_index("c")

  def _zero(i, carry):
    for k in range(B // 16):
      zbuf[i, pl.ds(k * 16, 16)] = jnp.zeros((16,), jnp.float32)
    return carry
  lax.fori_loop(0, ZROWS, _zero, 0)

  copies = []
  for t in range(ZPT):
    c = wid + t * NTILES
    off = jnp.minimum(c * ZROWS, ZLAST)
    off = pl.multiple_of(off, 8)
    copies.append(pltpu.async_copy(
        zbuf, out_hbm.at[pl.ds(off, ZROWS), :], sem))
  for c in copies:
    c.wait()


def _sc_fix_body(zeros_hbm, idx_hbm, out_hbm, idx_v, tbuf, sem):
  del zeros_hbm
  del sem
  wid = lax.axis_index("s") * 2 + lax.axis_index("c")
  lo = wid * OWN
  hi = jnp.minimum(lo + OWN, N)
  pltpu.sync_copy(idx_hbm, idx_v)
  lanes = lax.broadcasted_iota(jnp.int32, (16,), 0)

  def _entry(j, carry):
    j16 = pl.multiple_of(j // 16 * 16, 16)
    chunk = idx_v[0, pl.ds(j16, 16)]
    s_f = jnp.max(jnp.where(lanes == j - j16, chunk.astype(jnp.float32),
                            jnp.float32(-1.0)))
    s = s_f.astype(jnp.int32)

    @pl.when((s >= lo) & (s < hi))
    def _hit():
      r0 = pl.multiple_of(lax.bitwise_and(s, jnp.int32(-8)), 8)
      dst = out_hbm.at[pl.ds(r0, 8), :]
      pltpu.sync_copy(dst, tbuf)
      l0 = pl.multiple_of(j16, 16)
      v = tbuf[s - r0, pl.ds(l0, 16)]
      tbuf[s - r0, pl.ds(l0, 16)] = jnp.where(lanes == j - j16,
                                              jnp.float32(1.0), v)
      pltpu.sync_copy(tbuf, dst)
    return carry

  lax.fori_loop(0, B, _entry, 0)


@functools.cache
def _get_sc_calls():
  mesh = plsc.VectorSubcoreMesh(core_axis_name="c", subcore_axis_name="s")
  params = pltpu.CompilerParams(needs_layout_passes=False)
  zero_call = _pl_mpmd._mpmd_map(
      [(mesh, _sc_zero_body)],
      jax.ShapeDtypeStruct((N, B), jnp.float32),
      compiler_params=params,
      scratch_types=[
          pltpu.VMEM((ZROWS, B), jnp.float32),
          pltpu.SemaphoreType.DMA,
      ],
  )
  fix_call = _pl_mpmd._mpmd_map(
      [(mesh, _sc_fix_body)],
      jax.ShapeDtypeStruct((N, B), jnp.float32),
      input_output_aliases={0: 0},
      compiler_params=params,
      scratch_types=[
          pltpu.VMEM((1, B), jnp.int32),
          pltpu.VMEM((8, B), jnp.float32),
          pltpu.SemaphoreType.DMA,
      ],
  )
  return zero_call, fix_call


def kernel(sample):
  zero_call, fix_call = _get_sc_calls()
  xt = sample.T
  idx = _tc_call(xt)[0]
  zeros = zero_call()
  out_t = fix_call(zeros, idx)
  return out_t.T
